# baseline (speedup 1.0000x reference)
import functools

import jax
import jax.numpy as jnp
from jax import lax
from jax.experimental import pallas as pl
from jax.experimental.pallas import tpu as pltpu
from jax.experimental.pallas import tpu_sc as plsc

_EMBED = 32
_SCALE = float(_EMBED ** 0.5)

_PACK = 4
_W = _PACK * 128


def _gather_rows(table, x4t):
    njg, nig, njj, lanes = x4t.shape
    s = njg * njj
    b = nig * lanes
    n = s * b
    cw_per_j = b // _W
    n_workers = 32
    wpw = (s * cw_per_j) // n_workers
    mesh = plsc.VectorSubcoreMesh(core_axis_name="core",
                                  subcore_axis_name="subcore")

    @functools.partial(
        pl.kernel,
        out_type=jax.ShapeDtypeStruct((n, _EMBED), jnp.float32),
        mesh=mesh,
        compiler_params=pltpu.CompilerParams(use_tc_tiling_on_sc=False,
                                             needs_layout_passes=False),
        scratch_types=[
            pltpu.VMEM((2, _PACK, lanes), jnp.int32),
            pltpu.VMEM((2, _W), jnp.int32),
            pltpu.VMEM((2, _W, _EMBED), jnp.float32),
            pltpu.SemaphoreType.DMA,
            pltpu.SemaphoreType.DMA,
            pltpu.SemaphoreType.DMA,
        ],
    )
    def k(tab_hbm, i_hbm, o_hbm, idx_v, idx_s, rows_v, isem, gsem0, gsem1):
        wkr = lax.axis_index("core") * 16 + lax.axis_index("subcore")
        base_w = wkr * wpw
        gsems = (gsem0, gsem1)

        def load_idx(wg, buf):
            j = wg // cw_per_j
            cw = wg % cw_per_j
            jg = j // njj
            jj = j % njj
            for a in range(_PACK):
                pltpu.async_copy(i_hbm.at[jg, cw_per_j * a + cw, jj],
                                 idx_v.at[buf, a], isem)
            for a in range(_PACK):
                pltpu.make_async_copy(i_hbm.at[jg, cw_per_j * a + cw, jj],
                                      idx_v.at[buf, a], isem).wait()

        def permute(buf):
            src = idx_v.at[buf]
            for q in range(_W // 16):
                l = lax.iota(jnp.int32, 16)
                row = lax.rem(l, _PACK)
                col = (_PACK * q) + lax.div(l, _PACK)
                idx_s[buf, pl.ds(16 * q, 16)] = \
                    plsc.load_gather(src, [row, col])

        def fire(buf):
            for g in range(_PACK):
                pltpu.async_copy(
                    tab_hbm.at[idx_s.at[buf, pl.ds(g * lanes, lanes)]],
                    rows_v.at[buf, pl.ds(g * lanes, lanes)], gsems[buf])

        def drain(buf):
            for g in range(_PACK):
                pltpu.make_async_copy(
                    tab_hbm.at[idx_s.at[buf, pl.ds(g * lanes, lanes)]],
                    rows_v.at[buf, pl.ds(g * lanes, lanes)],
                    gsems[buf]).wait()

        def store(wg, buf):
            pltpu.sync_copy(rows_v.at[buf], o_hbm.at[pl.ds(wg * _W, _W)])

        load_idx(base_w, 0)
        permute(0)
        fire(0)

        @pl.loop(0, wpw)
        def _(c):
            wg = base_w + c

            @pl.when(c + 1 < wpw)
            def _():
                @pl.when((c + 1) % 2 == 0)
                def _():
                    load_idx(wg + 1, 0)
                    permute(0)
                    fire(0)

                @pl.when((c + 1) % 2 == 1)
                def _():
                    load_idx(wg + 1, 1)
                    permute(1)
                    fire(1)

            @pl.when(c % 2 == 0)
            def _():
                drain(0)
                store(wg, 0)

            @pl.when(c % 2 == 1)
            def _():
                drain(1)
                store(wg, 1)

    return k(table, x4t)


def _transpose_scale(glin, s, b):
    pack = 128 // _EMBED
    g4 = glin.reshape(s, b * _EMBED // 128, 128)
    chunk = b // pack

    def body(t_ref, o_ref):
        tv = jnp.swapaxes(t_ref[0], 0, 1) * _SCALE
        for a in range(pack):
            o_ref[0, :, a * chunk:(a + 1) * chunk] = \
                tv[a * _EMBED:(a + 1) * _EMBED, :]

    return pl.pallas_call(
        body,
        out_shape=jax.ShapeDtypeStruct((s, _EMBED, b), jnp.float32),
        grid=(s,),
        in_specs=[pl.BlockSpec((1, b * _EMBED // 128, 128),
                               lambda j: (j, 0, 0))],
        out_specs=pl.BlockSpec((1, _EMBED, b), lambda j: (j, 0, 0)),
        compiler_params=pltpu.CompilerParams(
            dimension_semantics=("parallel",)),
    )(g4)


def kernel(x, table):
    b, s = x.shape
    n = b * s
    xt = jnp.swapaxes(x, 0, 1)
    x4t = jnp.transpose(xt.reshape(s // 8, 8, b // 128, 128), (0, 2, 1, 3))
    g = _gather_rows(table, x4t)
    outp = _transpose_scale(g.reshape(n * _EMBED), s, b)
    return jnp.transpose(outp, (2, 0, 1))


# speedup vs baseline: 1.1128x; 1.1128x over previous
import functools

import jax
import jax.numpy as jnp
from jax import lax
from jax.experimental import pallas as pl
from jax.experimental.pallas import tpu as pltpu
from jax.experimental.pallas import tpu_sc as plsc

_EMBED = 32
_SCALE = float(_EMBED ** 0.5)

_PACK = 4
_W = _PACK * 128


def _gather_rows(table, x4t, chunk, n_chunks):
    njg, nig, njj, lanes = x4t.shape
    s = njg * njj
    b = nig * lanes
    n = s * b // n_chunks
    cw_per_j = b // _W
    n_workers = 32
    total_w = s * cw_per_j
    chunk_base = chunk * (total_w // n_chunks)
    wpw = total_w // (n_chunks * n_workers)
    mesh = plsc.VectorSubcoreMesh(core_axis_name="core",
                                  subcore_axis_name="subcore")

    @functools.partial(
        pl.kernel,
        out_type=jax.ShapeDtypeStruct((n, _EMBED), jnp.float32),
        mesh=mesh,
        compiler_params=pltpu.CompilerParams(use_tc_tiling_on_sc=False,
                                             needs_layout_passes=False),
        scratch_types=[
            pltpu.VMEM((2, _PACK, lanes), jnp.int32),
            pltpu.VMEM((2, _W), jnp.int32),
            pltpu.VMEM((2, _W, _EMBED), jnp.float32),
            pltpu.SemaphoreType.DMA,
            pltpu.SemaphoreType.DMA,
            pltpu.SemaphoreType.DMA,
        ],
    )
    def k(tab_hbm, i_hbm, o_hbm, idx_v, idx_s, rows_v, isem, gsem0, gsem1):
        wkr = lax.axis_index("core") * 16 + lax.axis_index("subcore")
        base_w = chunk_base + wkr * wpw
        gsems = (gsem0, gsem1)

        def load_idx(wg, buf):
            j = wg // cw_per_j
            cw = wg % cw_per_j
            jg = j // njj
            jj = j % njj
            for a in range(_PACK):
                pltpu.async_copy(i_hbm.at[jg, cw_per_j * a + cw, jj],
                                 idx_v.at[buf, a], isem)
            for a in range(_PACK):
                pltpu.make_async_copy(i_hbm.at[jg, cw_per_j * a + cw, jj],
                                      idx_v.at[buf, a], isem).wait()

        def permute(buf):
            src = idx_v.at[buf]
            for q in range(_W // 16):
                l = lax.iota(jnp.int32, 16)
                row = lax.rem(l, _PACK)
                col = (_PACK * q) + lax.div(l, _PACK)
                idx_s[buf, pl.ds(16 * q, 16)] = \
                    plsc.load_gather(src, [row, col])

        def fire(buf):
            for g in range(_PACK):
                pltpu.async_copy(
                    tab_hbm.at[idx_s.at[buf, pl.ds(g * lanes, lanes)]],
                    rows_v.at[buf, pl.ds(g * lanes, lanes)], gsems[buf])

        def drain(buf):
            for g in range(_PACK):
                pltpu.make_async_copy(
                    tab_hbm.at[idx_s.at[buf, pl.ds(g * lanes, lanes)]],
                    rows_v.at[buf, pl.ds(g * lanes, lanes)],
                    gsems[buf]).wait()

        def store(wg, buf):
            pltpu.sync_copy(rows_v.at[buf],
                            o_hbm.at[pl.ds((wg - chunk_base) * _W, _W)])

        load_idx(base_w, 0)
        permute(0)
        fire(0)

        @pl.loop(0, wpw)
        def _(c):
            wg = base_w + c

            @pl.when(c + 1 < wpw)
            def _():
                @pl.when((c + 1) % 2 == 0)
                def _():
                    load_idx(wg + 1, 0)
                    permute(0)
                    fire(0)

                @pl.when((c + 1) % 2 == 1)
                def _():
                    load_idx(wg + 1, 1)
                    permute(1)
                    fire(1)

            @pl.when(c % 2 == 0)
            def _():
                drain(0)
                store(wg, 0)

            @pl.when(c % 2 == 1)
            def _():
                drain(1)
                store(wg, 1)

    return k(table, x4t)


def _transpose_scale(glin, s, b, chunk, n_chunks, prev):
    pack = 128 // _EMBED
    sc_j = s // n_chunks
    base_j = chunk * sc_j
    g4 = glin.reshape(sc_j, b * _EMBED // 128, 128)
    cw = b // pack

    args = [g4]
    in_specs = [pl.BlockSpec((1, b * _EMBED // 128, 128),
                             lambda j: (j, 0, 0))]
    io_alias = {}
    if prev is not None:
        args.append(prev)
        in_specs.append(pl.BlockSpec(memory_space=pl.ANY))
        io_alias = {1: 0}

    def body(t_ref, *rest):
        o_ref = rest[-1]
        tv = jnp.swapaxes(t_ref[0], 0, 1) * _SCALE
        for a in range(pack):
            o_ref[0, :, a * cw:(a + 1) * cw] = \
                tv[a * _EMBED:(a + 1) * _EMBED, :]

    return pl.pallas_call(
        body,
        out_shape=jax.ShapeDtypeStruct((s, _EMBED, b), jnp.float32),
        grid=(sc_j,),
        in_specs=in_specs,
        out_specs=pl.BlockSpec((1, _EMBED, b),
                               lambda j: (base_j + j, 0, 0)),
        input_output_aliases=io_alias,
        compiler_params=pltpu.CompilerParams(
            dimension_semantics=("arbitrary",)),
    )(*args)


def kernel(x, table):
    b, s = x.shape
    n = b * s
    n_chunks = 5
    xt = jnp.swapaxes(x, 0, 1)
    x4t = jnp.transpose(xt.reshape(s // 8, 8, b // 128, 128), (0, 2, 1, 3))
    gs = [_gather_rows(table, x4t, c, n_chunks) for c in range(n_chunks)]
    outp = None
    for c in range(n_chunks):
        outp = _transpose_scale(gs[c].reshape(n * _EMBED // n_chunks),
                                s, b, c, n_chunks, outp)
    return jnp.transpose(outp, (2, 0, 1))
